# baseline (speedup 1.0000x reference)
import functools

import jax
import jax.numpy as jnp
from jax import lax
from jax.experimental import pallas as pl
from jax.experimental.pallas import tpu as pltpu
from jax.experimental.pallas import tpu_sc as plsc

N = 10000
E = 320000
D = 128

NC = 2
NS = 16
NW = NC * NS
NPAD = 10240
HALF = NPAD // 2
SEG = NPAD // NS

_CH0 = E // NW
_KQ = 16
_PADU = 2 * _KQ
_CAPW = 10240
_EP2 = NW * _CAPW


def _part_body(row_hbm, col_hbm, hist_out, rowall_out, colall_out, lens_out,
               rowv, colv, rhist, chist, bar, bac, bbr, bbc, tbuf, lenv,
               staging):
    c = lax.axis_index("c")
    s = lax.axis_index("s")
    w = s * NC + c

    zero16 = jnp.zeros((16,), jnp.float32)

    def zero(i, _):
        rhist[pl.ds(i * 16, 16)] = zero16
        chist[pl.ds(i * 16, 16)] = zero16
        return 0

    lax.fori_loop(0, NPAD // 16, zero, 0)

    pltpu.sync_copy(row_hbm.at[pl.ds(w * _CH0, _CH0)], rowv)
    pltpu.sync_copy(col_hbm.at[pl.ds(w * _CH0, _CH0)], colv)
    ones = jnp.ones((16,), jnp.float32)

    def part(i, tails):
        ta, tb = tails
        r16 = rowv[pl.ds(i * 16, 16)]
        c16 = colv[pl.ds(i * 16, 16)]
        plsc.addupdate_scatter(rhist, [r16], ones)
        plsc.addupdate_scatter(chist, [c16], ones)
        m = r16 < HALF
        na = plsc.all_reduce_population_count(m)[0]
        plsc.store_compressed(bar.at[pl.ds(ta, 16)], r16, mask=m)
        plsc.store_compressed(bac.at[pl.ds(ta, 16)], c16, mask=m)
        mb = jnp.logical_not(m)
        plsc.store_compressed(bbr.at[pl.ds(tb, 16)], r16 - HALF, mask=mb)
        plsc.store_compressed(bbc.at[pl.ds(tb, 16)], c16, mask=mb)
        return ta + na, tb + (16 - na)

    ta, tb = lax.fori_loop(0, _CH0 // 16, part, (jnp.int32(0), jnp.int32(0)))

    zrow16 = jnp.zeros((16,), jnp.int32)
    zcol16 = jnp.full((16,), N, jnp.int32)
    for j in range(_PADU // 16):
        bar[pl.ds(ta + j * 16, 16)] = zrow16
        bac[pl.ds(ta + j * 16, 16)] = zcol16
        bbr[pl.ds(tb + j * 16, 16)] = zrow16
        bbc[pl.ds(tb + j * 16, 16)] = zcol16
    nba = (ta + _PADU - 1) // _PADU * 2
    nbb = (tb + _PADU - 1) // _PADU * 2

    pltpu.sync_copy(bar, rowall_out.at[pl.ds(w * _CAPW, _CAPW)])
    pltpu.sync_copy(bac, colall_out.at[pl.ds(w * _CAPW, _CAPW)])
    pltpu.sync_copy(bbr, rowall_out.at[pl.ds(_EP2 + w * _CAPW, _CAPW)])
    pltpu.sync_copy(bbc, colall_out.at[pl.ds(_EP2 + w * _CAPW, _CAPW)])

    iot = lax.iota(jnp.int32, 16)
    lenv[pl.ds(0, 16)] = jnp.where(iot == 0, nba, 0)
    pltpu.sync_copy(lenv, lens_out.at[pl.ds(w * 16, 16)])
    lenv[pl.ds(0, 16)] = jnp.where(iot == 0, nbb, 0)
    pltpu.sync_copy(lenv, lens_out.at[pl.ds(NW * 16 + w * 16, 16)])

    def reduce_hist(hist, out_off):
        pltpu.sync_copy(hist, staging.at[s])
        plsc.subcore_barrier()

        def zero2(i, _):
            tbuf[pl.ds(i * 16, 16)] = zero16
            return 0

        lax.fori_loop(0, SEG // 16, zero2, 0)

        def red(t, _):
            pltpu.sync_copy(staging.at[t, pl.ds(s * SEG, SEG)],
                            rhist.at[pl.ds(0, SEG)])

            def vadd(j, _):
                tbuf[pl.ds(j * 16, 16)] = (tbuf[pl.ds(j * 16, 16)]
                                           + rhist[pl.ds(j * 16, 16)])
                return 0

            lax.fori_loop(0, SEG // 16, vadd, 0)
            return 0

        lax.fori_loop(0, NS, red, 0)
        pltpu.sync_copy(tbuf, hist_out.at[pl.ds(out_off + s * SEG, SEG)])
        plsc.subcore_barrier()

    reduce_hist(rhist, c * 2 * NPAD)
    reduce_hist(chist, c * 2 * NPAD + NPAD)


_part_kernel = functools.partial(
    pl.kernel,
    mesh=plsc.VectorSubcoreMesh(core_axis_name="c", subcore_axis_name="s"),
    compiler_params=pltpu.CompilerParams(needs_layout_passes=False),
    out_type=(
        jax.ShapeDtypeStruct((NC * 2 * NPAD,), jnp.float32),
        jax.ShapeDtypeStruct((2 * _EP2,), jnp.int32),
        jax.ShapeDtypeStruct((2 * _EP2,), jnp.int32),
        jax.ShapeDtypeStruct((2 * NW * 16,), jnp.int32),
    ),
    scratch_types=[
        pltpu.VMEM((_CH0,), jnp.int32),
        pltpu.VMEM((_CH0,), jnp.int32),
        pltpu.VMEM((NPAD,), jnp.float32),
        pltpu.VMEM((NPAD,), jnp.float32),
        pltpu.VMEM((_CAPW,), jnp.int32),
        pltpu.VMEM((_CAPW,), jnp.int32),
        pltpu.VMEM((_CAPW,), jnp.int32),
        pltpu.VMEM((_CAPW,), jnp.int32),
        pltpu.VMEM((SEG,), jnp.float32),
        pltpu.VMEM((16,), jnp.int32),
        pltpu.VMEM_SHARED((NS, NPAD), jnp.float32),
    ],
)(_part_body)

_RPT = HALF // NS


def _spmm_body(tabs_hbm, rowall_hbm, colall_hbm, lens_hbm, out_hbm,
               rowv0, rowv1, colv0, colv1, lenv, zbuf, buf0, buf1, acc,
               isem0, isem1, gsem0, gsem1):
    c = lax.axis_index("c")
    s = lax.axis_index("s")

    def zrow(i, _):
        def zlane(j, _):
            zbuf[i, 0, pl.ds(j * 16, 16)] = jnp.zeros((16,), jnp.float32)
            zbuf[i, 1, pl.ds(j * 16, 16)] = jnp.zeros((16,), jnp.float32)
            return 0

        lax.fori_loop(0, D // 16, zlane, 0)
        return 0

    lax.fori_loop(0, _KQ, zrow, 0)

    def zcp(t, _):
        pltpu.sync_copy(zbuf, acc.at[pl.ds(s * _RPT + t * _KQ, _KQ)])
        return 0

    lax.fori_loop(0, _RPT // _KQ, zcp, 0)
    plsc.subcore_barrier()

    for q in (0, 1):
        w = 2 * s + q
        base = c * _EP2 + w * _CAPW
        pltpu.sync_copy(lens_hbm.at[pl.ds(c * NW * 16 + w * 16, 16)], lenv)
        nb = lenv[pl.ds(0, 16)][0]

        def idx_copy(b, rowv, colv, isem):
            pltpu.async_copy(rowall_hbm.at[pl.ds(base + b * _KQ, _KQ)], rowv,
                             isem)
            pltpu.async_copy(colall_hbm.at[pl.ds(base + b * _KQ, _KQ)], colv,
                             isem)

        def idx_wait(b, rowv, colv, isem):
            pltpu.make_async_copy(rowall_hbm.at[pl.ds(base + b * _KQ, _KQ)],
                                  rowv, isem).wait()
            pltpu.make_async_copy(colall_hbm.at[pl.ds(base + b * _KQ, _KQ)],
                                  colv, isem).wait()

        def gat_copy(colv, buf, gsem):
            pltpu.async_copy(tabs_hbm.at[colv], buf, gsem)

        def gat_wait(colv, buf, gsem):
            pltpu.make_async_copy(tabs_hbm.at[colv], buf, gsem).wait()

        def scat(rowv, buf):
            pltpu.sync_copy(buf, acc.at[rowv], add=True)

        idx_copy(0, rowv0, colv0, isem0)
        idx_copy(1, rowv1, colv1, isem1)
        idx_wait(0, rowv0, colv0, isem0)
        gat_copy(colv0, buf0, gsem0)

        def pair(g, _):
            b0 = 2 * g
            b1 = b0 + 1
            idx_wait(b1, rowv1, colv1, isem1)
            gat_copy(colv1, buf1, gsem1)
            gat_wait(colv0, buf0, gsem0)
            scat(rowv0, buf0)
            idx_copy(b0 + 2, rowv0, colv0, isem0)
            idx_wait(b0 + 2, rowv0, colv0, isem0)
            gat_copy(colv0, buf0, gsem0)
            gat_wait(colv1, buf1, gsem1)
            scat(rowv1, buf1)
            idx_copy(b1 + 2, rowv1, colv1, isem1)
            return 0

        lax.fori_loop(0, nb // 2 - 1, pair, 0)
        bl = nb - 2
        idx_wait(bl + 1, rowv1, colv1, isem1)
        gat_copy(colv1, buf1, gsem1)
        gat_wait(colv0, buf0, gsem0)
        scat(rowv0, buf0)
        gat_wait(colv1, buf1, gsem1)
        scat(rowv1, buf1)

    plsc.subcore_barrier()
    pltpu.sync_copy(acc.at[pl.ds(s * _RPT, _RPT)],
                    out_hbm.at[pl.ds(c * HALF + s * _RPT, _RPT)])


_spmm_kernel = functools.partial(
    pl.kernel,
    mesh=plsc.VectorSubcoreMesh(core_axis_name="c", subcore_axis_name="s"),
    compiler_params=pltpu.CompilerParams(needs_layout_passes=False),
    out_type=jax.ShapeDtypeStruct((NPAD, 2, D), jnp.float32),
    scratch_types=[
        pltpu.VMEM((_KQ,), jnp.int32),
        pltpu.VMEM((_KQ,), jnp.int32),
        pltpu.VMEM((_KQ,), jnp.int32),
        pltpu.VMEM((_KQ,), jnp.int32),
        pltpu.VMEM((16,), jnp.int32),
        pltpu.VMEM((_KQ, 2, D), jnp.float32),
        pltpu.VMEM((_KQ, 2, D), jnp.float32),
        pltpu.VMEM((_KQ, 2, D), jnp.float32),
        pltpu.VMEM_SHARED((HALF, 2, D), jnp.float32),
        pltpu.SemaphoreType.DMA,
        pltpu.SemaphoreType.DMA,
        pltpu.SemaphoreType.DMA,
        pltpu.SemaphoreType.DMA,
    ],
)(_spmm_body)

_B2 = 2048


def _prescale_body(x_ref, dr0_ref, dr1_ref, dc0_ref, dc1_ref,
                   tabs_ref, invr_ref):
    x = x_ref[...]
    dr = dr0_ref[...] + dr1_ref[...]
    dc = dc0_ref[...] + dc1_ref[...]
    invr_ref[...] = lax.rsqrt(jnp.maximum(dr, 1.0))
    invc = lax.rsqrt(jnp.maximum(dc, 1.0))
    xs = x * invc
    tabs_ref[...] = jnp.concatenate([xs, xs * x], axis=1)


def _prescale(xp, dr0, dr1, dc0, dc1):
    return pl.pallas_call(
        _prescale_body,
        grid=(NPAD // _B2,),
        in_specs=[
            pl.BlockSpec((_B2, D), lambda i: (i, 0)),
            pl.BlockSpec((_B2, 1), lambda i: (i, 0)),
            pl.BlockSpec((_B2, 1), lambda i: (i, 0)),
            pl.BlockSpec((_B2, 1), lambda i: (i, 0)),
            pl.BlockSpec((_B2, 1), lambda i: (i, 0)),
        ],
        out_specs=[
            pl.BlockSpec((_B2, 2 * D), lambda i: (i, 0)),
            pl.BlockSpec((_B2, 1), lambda i: (i, 0)),
        ],
        out_shape=[
            jax.ShapeDtypeStruct((NPAD, 2 * D), jnp.float32),
            jax.ShapeDtypeStruct((NPAD, 1), jnp.float32),
        ],
    )(xp, dr0, dr1, dc0, dc1)


_B = 2000


def _finish_body(x_ref, a1_ref, a2_ref, invr_ref, w1_ref, w2_ref, b1_ref,
                 b2_ref, o_ref):
    invr = invr_ref[...]
    a = x_ref[...] + invr * a1_ref[...]
    b = invr * a2_ref[...]
    s = (jnp.dot(a, w1_ref[...], preferred_element_type=jnp.float32)
         + jnp.dot(b, w2_ref[...], preferred_element_type=jnp.float32)
         + 2.0 * b1_ref[...] + b2_ref[...])
    o_ref[...] = jnp.where(s >= 0, s, 0.2 * s)


def _finish(x, agg, invr, W1, W2, b1, b2):
    return pl.pallas_call(
        _finish_body,
        grid=(N // _B,),
        in_specs=[
            pl.BlockSpec((_B, D), lambda i: (i, 0)),
            pl.BlockSpec((_B, D), lambda i: (i, 0)),
            pl.BlockSpec((_B, D), lambda i: (i, 1)),
            pl.BlockSpec((_B, 1), lambda i: (i, 0)),
            pl.BlockSpec((D, D), lambda i: (0, 0)),
            pl.BlockSpec((D, D), lambda i: (0, 0)),
            pl.BlockSpec((1, D), lambda i: (0, 0)),
            pl.BlockSpec((1, D), lambda i: (0, 0)),
        ],
        out_specs=pl.BlockSpec((_B, D), lambda i: (i, 0)),
        out_shape=jax.ShapeDtypeStruct((N, D), jnp.float32),
    )(x, agg, agg, invr, W1, W2, b1, b2)


def kernel(edge_index, node_features, W1, b1, W2, b2):
    row = edge_index[0]
    col = edge_index[1]

    hist, rowall, colall, lens = _part_kernel(row, col)
    dr0 = hist[0:NPAD].reshape(NPAD, 1)
    dc0 = hist[NPAD:2 * NPAD].reshape(NPAD, 1)
    dr1 = hist[2 * NPAD:3 * NPAD].reshape(NPAD, 1)
    dc1 = hist[3 * NPAD:4 * NPAD].reshape(NPAD, 1)

    xp = jnp.pad(node_features, ((0, NPAD - N), (0, 0)))
    tabs, invr = _prescale(xp, dr0, dr1, dc0, dc1)

    agg = _spmm_kernel(tabs.reshape(NPAD, 2, D), rowall, colall, lens)

    return _finish(node_features, agg.reshape(NPAD, 2 * D), invr[:N], W1, W2,
                   b1.reshape(1, D), b2.reshape(1, D))


# speedup vs baseline: 1.4648x; 1.4648x over previous
import functools

import jax
import jax.numpy as jnp
from jax import lax
from jax.experimental import pallas as pl
from jax.experimental.pallas import tpu as pltpu
from jax.experimental.pallas import tpu_sc as plsc

N = 10000
E = 320000
D = 128

NC = 2
NS = 16
NW = NC * NS
NPAD = 10240
HALF = NPAD // 2
SEG = NPAD // NS

_CH0 = E // NW
_KQ = 32
_PADU = 2 * _KQ
_CAPW = 10240
_EP2 = NW * _CAPW


def _part_body(row_hbm, col_hbm, hist_out, rowall_out, colall_out, lens_out,
               rowv, colv, rhist, chist, bar, bac, bbr, bbc, tbuf, lenv,
               staging):
    c = lax.axis_index("c")
    s = lax.axis_index("s")
    w = s * NC + c

    zero16 = jnp.zeros((16,), jnp.float32)

    def zero(i, _):
        rhist[pl.ds(i * 16, 16)] = zero16
        chist[pl.ds(i * 16, 16)] = zero16
        return 0

    lax.fori_loop(0, NPAD // 16, zero, 0)

    pltpu.sync_copy(row_hbm.at[pl.ds(w * _CH0, _CH0)], rowv)
    pltpu.sync_copy(col_hbm.at[pl.ds(w * _CH0, _CH0)], colv)
    ones = jnp.ones((16,), jnp.float32)

    def part(i, tails):
        ta, tb = tails
        r16 = rowv[pl.ds(i * 16, 16)]
        c16 = colv[pl.ds(i * 16, 16)]
        plsc.addupdate_scatter(rhist, [r16], ones)
        plsc.addupdate_scatter(chist, [c16], ones)
        m = r16 < HALF
        na = plsc.all_reduce_population_count(m)[0]
        plsc.store_compressed(bar.at[pl.ds(ta, 16)], r16, mask=m)
        plsc.store_compressed(bac.at[pl.ds(ta, 16)], c16, mask=m)
        mb = jnp.logical_not(m)
        plsc.store_compressed(bbr.at[pl.ds(tb, 16)], r16 - HALF, mask=mb)
        plsc.store_compressed(bbc.at[pl.ds(tb, 16)], c16, mask=mb)
        return ta + na, tb + (16 - na)

    ta, tb = lax.fori_loop(0, _CH0 // 16, part, (jnp.int32(0), jnp.int32(0)))

    zrow16 = jnp.zeros((16,), jnp.int32)
    zcol16 = jnp.full((16,), N, jnp.int32)
    for j in range(_PADU // 16):
        bar[pl.ds(ta + j * 16, 16)] = zrow16
        bac[pl.ds(ta + j * 16, 16)] = zcol16
        bbr[pl.ds(tb + j * 16, 16)] = zrow16
        bbc[pl.ds(tb + j * 16, 16)] = zcol16
    nba = (ta + _PADU - 1) // _PADU * 2
    nbb = (tb + _PADU - 1) // _PADU * 2

    pltpu.sync_copy(bar, rowall_out.at[pl.ds(w * _CAPW, _CAPW)])
    pltpu.sync_copy(bac, colall_out.at[pl.ds(w * _CAPW, _CAPW)])
    pltpu.sync_copy(bbr, rowall_out.at[pl.ds(_EP2 + w * _CAPW, _CAPW)])
    pltpu.sync_copy(bbc, colall_out.at[pl.ds(_EP2 + w * _CAPW, _CAPW)])

    iot = lax.iota(jnp.int32, 16)
    lenv[pl.ds(0, 16)] = jnp.where(iot == 0, nba, 0)
    pltpu.sync_copy(lenv, lens_out.at[pl.ds(w * 16, 16)])
    lenv[pl.ds(0, 16)] = jnp.where(iot == 0, nbb, 0)
    pltpu.sync_copy(lenv, lens_out.at[pl.ds(NW * 16 + w * 16, 16)])

    def reduce_hist(hist, out_off):
        pltpu.sync_copy(hist, staging.at[s])
        plsc.subcore_barrier()

        def zero2(i, _):
            tbuf[pl.ds(i * 16, 16)] = zero16
            return 0

        lax.fori_loop(0, SEG // 16, zero2, 0)

        def red(t, _):
            pltpu.sync_copy(staging.at[t, pl.ds(s * SEG, SEG)],
                            rhist.at[pl.ds(0, SEG)])

            def vadd(j, _):
                tbuf[pl.ds(j * 16, 16)] = (tbuf[pl.ds(j * 16, 16)]
                                           + rhist[pl.ds(j * 16, 16)])
                return 0

            lax.fori_loop(0, SEG // 16, vadd, 0)
            return 0

        lax.fori_loop(0, NS, red, 0)
        pltpu.sync_copy(tbuf, hist_out.at[pl.ds(out_off + s * SEG, SEG)])
        plsc.subcore_barrier()

    reduce_hist(rhist, c * 2 * NPAD)
    reduce_hist(chist, c * 2 * NPAD + NPAD)


_part_kernel = functools.partial(
    pl.kernel,
    mesh=plsc.VectorSubcoreMesh(core_axis_name="c", subcore_axis_name="s"),
    compiler_params=pltpu.CompilerParams(needs_layout_passes=False),
    out_type=(
        jax.ShapeDtypeStruct((NC * 2 * NPAD,), jnp.float32),
        jax.ShapeDtypeStruct((2 * _EP2,), jnp.int32),
        jax.ShapeDtypeStruct((2 * _EP2,), jnp.int32),
        jax.ShapeDtypeStruct((2 * NW * 16,), jnp.int32),
    ),
    scratch_types=[
        pltpu.VMEM((_CH0,), jnp.int32),
        pltpu.VMEM((_CH0,), jnp.int32),
        pltpu.VMEM((NPAD,), jnp.float32),
        pltpu.VMEM((NPAD,), jnp.float32),
        pltpu.VMEM((_CAPW,), jnp.int32),
        pltpu.VMEM((_CAPW,), jnp.int32),
        pltpu.VMEM((_CAPW,), jnp.int32),
        pltpu.VMEM((_CAPW,), jnp.int32),
        pltpu.VMEM((SEG,), jnp.float32),
        pltpu.VMEM((16,), jnp.int32),
        pltpu.VMEM_SHARED((NS, NPAD), jnp.float32),
    ],
)(_part_body)

_RPT = HALF // NS


def _spmm_body(tabs_hbm, rowall_hbm, colall_hbm, lens_hbm, out_hbm,
               rowv0, rowv1, colv0, colv1, rowvn0, colvn0, rowvn1, colvn1,
               lenv, zbuf, buf0, buf1, acc,
               isem0, isem1, isemn0, isemn1, gsem0, gsem1):
    c = lax.axis_index("c")
    s = lax.axis_index("s")

    def zrow(i, _):
        def zlane(j, _):
            zbuf[i, 0, pl.ds(j * 16, 16)] = jnp.zeros((16,), jnp.float32)
            zbuf[i, 1, pl.ds(j * 16, 16)] = jnp.zeros((16,), jnp.float32)
            return 0

        lax.fori_loop(0, D // 16, zlane, 0)
        return 0

    lax.fori_loop(0, _KQ, zrow, 0)

    def zcp(t, _):
        pltpu.sync_copy(zbuf, acc.at[pl.ds(s * _RPT + t * _KQ, _KQ)])
        return 0

    lax.fori_loop(0, _RPT // _KQ, zcp, 0)
    plsc.subcore_barrier()

    for q in (0, 1):
        w = 2 * s + q
        base = c * _EP2 + w * _CAPW
        pltpu.sync_copy(lens_hbm.at[pl.ds(c * NW * 16 + w * 16, 16)], lenv)
        nb = lenv[pl.ds(0, 16)][0]

        def idx_copy(b, rowv, colv, isem):
            pltpu.async_copy(rowall_hbm.at[pl.ds(base + b * _KQ, _KQ)], rowv,
                             isem)
            pltpu.async_copy(colall_hbm.at[pl.ds(base + b * _KQ, _KQ)], colv,
                             isem)

        def idx_wait(b, rowv, colv, isem):
            pltpu.make_async_copy(rowall_hbm.at[pl.ds(base + b * _KQ, _KQ)],
                                  rowv, isem).wait()
            pltpu.make_async_copy(colall_hbm.at[pl.ds(base + b * _KQ, _KQ)],
                                  colv, isem).wait()

        def gat_copy(colv, buf, gsem):
            pltpu.async_copy(tabs_hbm.at[colv], buf, gsem)

        def gat_wait(colv, buf, gsem):
            pltpu.make_async_copy(tabs_hbm.at[colv], buf, gsem).wait()

        def scat(rowv, buf):
            pltpu.sync_copy(buf, acc.at[rowv], add=True)

        def vcp(dstr, dstc, srcr, srcc):
            for j in range(_KQ // 16):
                dstr[pl.ds(j * 16, 16)] = srcr[pl.ds(j * 16, 16)]
                dstc[pl.ds(j * 16, 16)] = srcc[pl.ds(j * 16, 16)]

        idx_copy(0, rowv0, colv0, isem0)
        idx_copy(1, rowv1, colv1, isem1)
        idx_copy(2, rowvn0, colvn0, isemn0)
        idx_copy(3, rowvn1, colvn1, isemn1)
        idx_wait(0, rowv0, colv0, isem0)
        gat_copy(colv0, buf0, gsem0)
        idx_wait(1, rowv1, colv1, isem1)

        def pair(g, _):
            b0 = 2 * g
            b1 = b0 + 1
            gat_copy(colv1, buf1, gsem1)
            gat_wait(colv0, buf0, gsem0)
            scat(rowv0, buf0)
            idx_wait(b0 + 2, rowvn0, colvn0, isemn0)
            vcp(rowv0, colv0, rowvn0, colvn0)
            idx_copy(b0 + 4, rowvn0, colvn0, isemn0)
            gat_copy(colv0, buf0, gsem0)
            gat_wait(colv1, buf1, gsem1)
            scat(rowv1, buf1)
            idx_wait(b1 + 2, rowvn1, colvn1, isemn1)
            vcp(rowv1, colv1, rowvn1, colvn1)
            idx_copy(b1 + 4, rowvn1, colvn1, isemn1)
            return 0

        lax.fori_loop(0, nb // 2 - 1, pair, 0)
        gat_copy(colv1, buf1, gsem1)
        gat_wait(colv0, buf0, gsem0)
        scat(rowv0, buf0)
        gat_wait(colv1, buf1, gsem1)
        scat(rowv1, buf1)
        idx_wait(nb, rowvn0, colvn0, isemn0)
        idx_wait(nb + 1, rowvn1, colvn1, isemn1)

    plsc.subcore_barrier()
    pltpu.sync_copy(acc.at[pl.ds(s * _RPT, _RPT)],
                    out_hbm.at[pl.ds(c * HALF + s * _RPT, _RPT)])


_spmm_kernel = functools.partial(
    pl.kernel,
    mesh=plsc.VectorSubcoreMesh(core_axis_name="c", subcore_axis_name="s"),
    compiler_params=pltpu.CompilerParams(needs_layout_passes=False),
    out_type=jax.ShapeDtypeStruct((NPAD, 2, D), jnp.float32),
    scratch_types=[
        pltpu.VMEM((_KQ,), jnp.int32),
        pltpu.VMEM((_KQ,), jnp.int32),
        pltpu.VMEM((_KQ,), jnp.int32),
        pltpu.VMEM((_KQ,), jnp.int32),
        pltpu.VMEM((_KQ,), jnp.int32),
        pltpu.VMEM((_KQ,), jnp.int32),
        pltpu.VMEM((_KQ,), jnp.int32),
        pltpu.VMEM((_KQ,), jnp.int32),
        pltpu.VMEM((16,), jnp.int32),
        pltpu.VMEM((_KQ, 2, D), jnp.float32),
        pltpu.VMEM((_KQ, 2, D), jnp.float32),
        pltpu.VMEM((_KQ, 2, D), jnp.float32),
        pltpu.VMEM_SHARED((HALF, 2, D), jnp.float32),
        pltpu.SemaphoreType.DMA,
        pltpu.SemaphoreType.DMA,
        pltpu.SemaphoreType.DMA,
        pltpu.SemaphoreType.DMA,
        pltpu.SemaphoreType.DMA,
        pltpu.SemaphoreType.DMA,
    ],
)(_spmm_body)

_B2 = 2048


def _prescale_body(x_ref, dr0_ref, dr1_ref, dc0_ref, dc1_ref,
                   tabs_ref, invr_ref):
    x = x_ref[...]
    dr = dr0_ref[...] + dr1_ref[...]
    dc = dc0_ref[...] + dc1_ref[...]
    invr_ref[...] = lax.rsqrt(jnp.maximum(dr, 1.0))
    invc = lax.rsqrt(jnp.maximum(dc, 1.0))
    xs = x * invc
    tabs_ref[...] = jnp.concatenate([xs, xs * x], axis=1)


def _prescale(xp, dr0, dr1, dc0, dc1):
    return pl.pallas_call(
        _prescale_body,
        grid=(NPAD // _B2,),
        in_specs=[
            pl.BlockSpec((_B2, D), lambda i: (i, 0)),
            pl.BlockSpec((_B2, 1), lambda i: (i, 0)),
            pl.BlockSpec((_B2, 1), lambda i: (i, 0)),
            pl.BlockSpec((_B2, 1), lambda i: (i, 0)),
            pl.BlockSpec((_B2, 1), lambda i: (i, 0)),
        ],
        out_specs=[
            pl.BlockSpec((_B2, 2 * D), lambda i: (i, 0)),
            pl.BlockSpec((_B2, 1), lambda i: (i, 0)),
        ],
        out_shape=[
            jax.ShapeDtypeStruct((NPAD, 2 * D), jnp.float32),
            jax.ShapeDtypeStruct((NPAD, 1), jnp.float32),
        ],
    )(xp, dr0, dr1, dc0, dc1)


_B = 2000


def _finish_body(x_ref, a1_ref, a2_ref, invr_ref, w1_ref, w2_ref, b1_ref,
                 b2_ref, o_ref):
    invr = invr_ref[...]
    a = x_ref[...] + invr * a1_ref[...]
    b = invr * a2_ref[...]
    s = (jnp.dot(a, w1_ref[...], preferred_element_type=jnp.float32)
         + jnp.dot(b, w2_ref[...], preferred_element_type=jnp.float32)
         + 2.0 * b1_ref[...] + b2_ref[...])
    o_ref[...] = jnp.where(s >= 0, s, 0.2 * s)


def _finish(x, agg, invr, W1, W2, b1, b2):
    return pl.pallas_call(
        _finish_body,
        grid=(N // _B,),
        in_specs=[
            pl.BlockSpec((_B, D), lambda i: (i, 0)),
            pl.BlockSpec((_B, D), lambda i: (i, 0)),
            pl.BlockSpec((_B, D), lambda i: (i, 1)),
            pl.BlockSpec((_B, 1), lambda i: (i, 0)),
            pl.BlockSpec((D, D), lambda i: (0, 0)),
            pl.BlockSpec((D, D), lambda i: (0, 0)),
            pl.BlockSpec((1, D), lambda i: (0, 0)),
            pl.BlockSpec((1, D), lambda i: (0, 0)),
        ],
        out_specs=pl.BlockSpec((_B, D), lambda i: (i, 0)),
        out_shape=jax.ShapeDtypeStruct((N, D), jnp.float32),
    )(x, agg, agg, invr, W1, W2, b1, b2)


def kernel(edge_index, node_features, W1, b1, W2, b2):
    row = edge_index[0]
    col = edge_index[1]

    hist, rowall, colall, lens = _part_kernel(row, col)
    dr0 = hist[0:NPAD].reshape(NPAD, 1)
    dc0 = hist[NPAD:2 * NPAD].reshape(NPAD, 1)
    dr1 = hist[2 * NPAD:3 * NPAD].reshape(NPAD, 1)
    dc1 = hist[3 * NPAD:4 * NPAD].reshape(NPAD, 1)

    xp = jnp.pad(node_features, ((0, NPAD - N), (0, 0)))
    tabs, invr = _prescale(xp, dr0, dr1, dc0, dc1)

    agg = _spmm_kernel(tabs.reshape(NPAD, 2, D), rowall, colall, lens)

    return _finish(node_features, agg.reshape(NPAD, 2 * D), invr[:N], W1, W2,
                   b1.reshape(1, D), b2.reshape(1, D))
